# Initial kernel scaffold; baseline (speedup 1.0000x reference)
import jax
import jax.numpy as jnp
from jax.experimental import pallas as pl


def kernel(x, edge_u_x, edge_u_id, edge_index, ae_enc_in_W, ae_enc_in_b, ae_henc0_W, ae_henc0_b, ae_henc1_W, ae_henc1_b, ae_z_W, ae_z_b, ae_dec_in_W, ae_dec_in_b, ae_hdec0_W, ae_hdec0_b, ae_hdec1_W, ae_hdec1_b, ae_xbar_W, ae_xbar_b, gnn_in_W, gnn_in_b, gnn_h0_W, gnn_h0_b, gnn_h1_W, gnn_h1_b, gnn_nz_W, gnn_nz_b, gnn_cl_W, gnn_cl_b, cluster_layer):
    raise NotImplementedError("write your pallas kernel here")


# speedup vs baseline: 2.8081x; 2.8081x over previous
import functools

import jax
import jax.numpy as jnp
from jax import lax
from jax.experimental import pallas as pl
from jax.experimental.pallas import tpu as pltpu
from jax.experimental.pallas import tpu_sc as plsc

N = 10000
NPAD = 10240
E = 160000
EPAD = 163840
U = 4000
D_IN = 256
D_ENC = 512
D_Z = 64
D_CL = 32
R = 400
F32 = jnp.float32


_MESH = plsc.VectorSubcoreMesh(
    core_axis_name="c", subcore_axis_name="s", num_cores=2, num_subcores=16)


def _deg_kernel():
  @functools.partial(
      pl.kernel,
      out_type=jax.ShapeDtypeStruct((2 * NPAD, 128), F32),
      mesh=_MESH,
      scratch_types=[
          pltpu.VMEM_SHARED((NPAD, 128), F32),
          pltpu.VMEM((40, 128), jnp.int32),
          pltpu.VMEM((128, 128), F32),
      ],
  )
  def k(dst32, e0, zeros128, out, acc, didx, gbuf):
    cid = lax.axis_index("c")
    sid = lax.axis_index("s")
    w = cid * 16 + sid
    pltpu.sync_copy(e0, gbuf)
    pltpu.sync_copy(dst32.at[w], didx)
    pltpu.sync_copy(zeros128, acc.at[pl.ds(sid * 640, 640)])
    plsc.subcore_barrier()

    def body(j, carry):
      pltpu.sync_copy(gbuf, acc.at[didx.at[j]], add=True)
      return carry

    lax.fori_loop(0, 40, body, 0)
    plsc.subcore_barrier()
    pltpu.sync_copy(acc.at[pl.ds(sid * 640, 640)],
                    out.at[pl.ds(cid * NPAD + sid * 640, 640)])

  return k


def _agg_slice_kernel(C):
  halfc = C // 2

  @functools.partial(
      pl.kernel,
      out_type=jax.ShapeDtypeStruct((C * NPAD, 128), F32),
      mesh=_MESH,
      scratch_types=[
          pltpu.VMEM_SHARED((NPAD, 128), F32),
          pltpu.VMEM((80, 128), jnp.int32),
          pltpu.VMEM((80, 128), jnp.int32),
          pltpu.VMEM((128, 128), F32),
          pltpu.SemaphoreType.DMA,
      ],
  )
  def k(table, srcr, dst16, zeros128, out, acc, sidx, didx, gbuf, sem):
    cid = lax.axis_index("c")
    sid = lax.axis_index("s")
    pltpu.sync_copy(dst16.at[sid], didx)
    for kk in range(halfc):
      sl = cid * halfc + kk
      pltpu.sync_copy(srcr.at[sl, sid], sidx)
      pltpu.sync_copy(zeros128, acc.at[pl.ds(sid * 640, 640)])
      plsc.subcore_barrier()

      def body(j, carry):
        pltpu.async_copy(table.at[sidx.at[j]], gbuf, sem).wait()
        pltpu.sync_copy(gbuf, acc.at[didx.at[j]], add=True)
        return carry

      lax.fori_loop(0, 80, body, 0)
      plsc.subcore_barrier()
      pltpu.sync_copy(acc.at[pl.ds(sid * 640, 640)],
                      out.at[pl.ds(sl * NPAD + sid * 640, 640)])
      if kk + 1 < halfc:
        plsc.subcore_barrier()

  return k


def _agg_narrow_kernel():
  W = 128

  @functools.partial(
      pl.kernel,
      out_type=jax.ShapeDtypeStruct((2 * NPAD, W), F32),
      mesh=_MESH,
      scratch_types=[
          pltpu.VMEM_SHARED((NPAD, W), F32),
          pltpu.VMEM((40, 128), jnp.int32),
          pltpu.VMEM((40, 128), jnp.int32),
          pltpu.VMEM((128, W), F32),
          pltpu.SemaphoreType.DMA,
      ],
  )
  def k(table, src32, dst32, zerosw, out, acc, sidx, didx, gbuf, sem):
    cid = lax.axis_index("c")
    sid = lax.axis_index("s")
    w = cid * 16 + sid
    pltpu.sync_copy(src32.at[w], sidx)
    pltpu.sync_copy(dst32.at[w], didx)
    pltpu.sync_copy(zerosw, acc.at[pl.ds(sid * 640, 640)])
    plsc.subcore_barrier()

    def body(j, carry):
      pltpu.async_copy(table.at[sidx.at[j]], gbuf, sem).wait()
      pltpu.sync_copy(gbuf, acc.at[didx.at[j]], add=True)
      return carry

    lax.fori_loop(0, 40, body, 0)
    plsc.subcore_barrier()
    pltpu.sync_copy(acc.at[pl.ds(sid * 640, 640)],
                    out.at[pl.ds(cid * NPAD + sid * 640, 640)])

  return k


_DEG = _deg_kernel()
_AGG2 = _agg_slice_kernel(2)
_AGG4 = _agg_slice_kernel(4)
_AGG1 = _agg_narrow_kernel()


def _dot(a, b):
  return jnp.dot(a, b, preferred_element_type=F32)


def _ae_body(exu, w0, b0, w1, b1, w2, b2, wz, bz, wd, bd, wh0, bh0, wh1, bh1,
             wx, bx, enc0_o, enc1_o, enc2_o, z_o, xbar_o):
  e0 = jnp.maximum(_dot(exu[...], w0[...]) + b0[...], 0.0)
  e1 = jnp.maximum(_dot(e0, w1[...]) + b1[...], 0.0)
  e2 = jnp.maximum(_dot(e1, w2[...]) + b2[...], 0.0)
  z = _dot(e2, wz[...]) + bz[...]
  d0 = jnp.maximum(_dot(z, wd[...]) + bd[...], 0.0)
  d1 = jnp.maximum(_dot(d0, wh0[...]) + bh0[...], 0.0)
  d2 = jnp.maximum(_dot(d1, wh1[...]) + bh1[...], 0.0)
  xb = _dot(d2, wx[...]) + bx[...]
  enc0_o[...] = e0
  enc1_o[...] = e1
  enc2_o[...] = e2
  z_o[...] = z
  xbar_o[...] = xb


def _ae_call(exu, ws):
  full = lambda shp: pl.BlockSpec(shp, lambda i: (0,) * len(shp))
  specs = [pl.BlockSpec((R, D_IN), lambda i: (i, 0))]
  for wname, din, dout in [
      ("ae_enc_in", D_IN, D_ENC), ("ae_henc0", D_ENC, D_ENC),
      ("ae_henc1", D_ENC, D_ENC), ("ae_z", D_ENC, D_Z),
      ("ae_dec_in", D_Z, D_ENC), ("ae_hdec0", D_ENC, D_ENC),
      ("ae_hdec1", D_ENC, D_ENC), ("ae_xbar", D_ENC, D_IN)]:
    specs.append(full((din, dout)))
    specs.append(full((dout,)))
  outs = [
      jax.ShapeDtypeStruct((U, D_ENC), F32),
      jax.ShapeDtypeStruct((U, D_ENC), F32),
      jax.ShapeDtypeStruct((U, D_ENC), F32),
      jax.ShapeDtypeStruct((U, D_Z), F32),
      jax.ShapeDtypeStruct((U, D_IN), F32),
  ]
  out_specs = [
      pl.BlockSpec((R, D_ENC), lambda i: (i, 0)),
      pl.BlockSpec((R, D_ENC), lambda i: (i, 0)),
      pl.BlockSpec((R, D_ENC), lambda i: (i, 0)),
      pl.BlockSpec((R, D_Z), lambda i: (i, 0)),
      pl.BlockSpec((R, D_IN), lambda i: (i, 0)),
  ]
  return pl.pallas_call(
      _ae_body, grid=(U // R,), in_specs=specs, out_specs=out_specs,
      out_shape=outs)(exu, *ws)


def _rdeg_of(degp):
  deg = degp[0, :, 0] + degp[1, :, 0]
  return 1.0 / jnp.maximum(deg, 1.0)


def _l1_call(x, agg1, degp, w, b, enc0):
  def body(x_ref, agg, degp_r, wt, bt, skip, out):
    i = pl.program_id(0)
    rdeg = _rdeg_of(degp_r[...])
    a = jnp.concatenate([agg[0], agg[1]], axis=1)
    o = jnp.maximum(
        _dot(x_ref[...] + a * rdeg[:, None], wt[...]) + bt[...], 0.0)
    o = o + jnp.where(i < U // R, 1.0, 0.0) * skip[...]
    for c in range(4):
      out[c] = o[:, c * 128:(c + 1) * 128]

  return pl.pallas_call(
      body, grid=(N // R,),
      in_specs=[
          pl.BlockSpec((R, D_IN), lambda i: (i, 0)),
          pl.BlockSpec((2, R, 128), lambda i: (0, i, 0)),
          pl.BlockSpec((2, R, 128), lambda i: (0, i, 0)),
          pl.BlockSpec((D_IN, D_ENC), lambda i: (0, 0)),
          pl.BlockSpec((D_ENC,), lambda i: (0,)),
          pl.BlockSpec((R, D_ENC), lambda i: (jnp.minimum(i, U // R - 1), 0)),
      ],
      out_specs=pl.BlockSpec((4, R, 128), lambda i: (0, i, 0)),
      out_shape=jax.ShapeDtypeStruct((4, N, 128), F32))(
          x, agg1, degp, w, b, enc0)


def _lmid_call(xg, agg, degp, w, b, skip):
  def body(x_refs, agg_r, degp_r, wt, bt, skip_r, out):
    i = pl.program_id(0)
    rdeg = _rdeg_of(degp_r[...])
    h = jnp.concatenate([x_refs[c] for c in range(4)], axis=1)
    a = jnp.concatenate([agg_r[c] for c in range(4)], axis=1)
    o = jnp.maximum(_dot(h + a * rdeg[:, None], wt[...]) + bt[...], 0.0)
    o = o + jnp.where(i < U // R, 1.0, 0.0) * skip_r[...]
    for c in range(4):
      out[c] = o[:, c * 128:(c + 1) * 128]

  return pl.pallas_call(
      body, grid=(N // R,),
      in_specs=[
          pl.BlockSpec((4, R, 128), lambda i: (0, i, 0)),
          pl.BlockSpec((4, R, 128), lambda i: (0, i, 0)),
          pl.BlockSpec((2, R, 128), lambda i: (0, i, 0)),
          pl.BlockSpec((D_ENC, D_ENC), lambda i: (0, 0)),
          pl.BlockSpec((D_ENC,), lambda i: (0,)),
          pl.BlockSpec((R, D_ENC), lambda i: (jnp.minimum(i, U // R - 1), 0)),
      ],
      out_specs=pl.BlockSpec((4, R, 128), lambda i: (0, i, 0)),
      out_shape=jax.ShapeDtypeStruct((4, N, 128), F32))(
          xg, agg, degp, w, b, skip)


def _l3_call(xg, agg, degp, w, b, skip, w_nz):
  def body(x_refs, agg_r, degp_r, wt, bt, skip_r, wnz, out):
    i = pl.program_id(0)
    rdeg = _rdeg_of(degp_r[...])
    h = jnp.concatenate([x_refs[c] for c in range(4)], axis=1)
    a = jnp.concatenate([agg_r[c] for c in range(4)], axis=1)
    o = jnp.maximum(_dot(h + a * rdeg[:, None], wt[...]) + bt[...], 0.0)
    o = o + jnp.where(i < U // R, 1.0, 0.0) * skip_r[...]
    y4 = _dot(o, wnz[...])
    out[...] = jnp.concatenate([y4, jnp.zeros((R, 128 - D_Z), F32)], axis=1)

  return pl.pallas_call(
      body, grid=(N // R,),
      in_specs=[
          pl.BlockSpec((4, R, 128), lambda i: (0, i, 0)),
          pl.BlockSpec((4, R, 128), lambda i: (0, i, 0)),
          pl.BlockSpec((2, R, 128), lambda i: (0, i, 0)),
          pl.BlockSpec((D_ENC, D_ENC), lambda i: (0, 0)),
          pl.BlockSpec((D_ENC,), lambda i: (0,)),
          pl.BlockSpec((R, D_ENC), lambda i: (jnp.minimum(i, U // R - 1), 0)),
          pl.BlockSpec((D_ENC, D_Z), lambda i: (0, 0)),
      ],
      out_specs=pl.BlockSpec((R, 128), lambda i: (i, 0)),
      out_shape=jax.ShapeDtypeStruct((N, 128), F32))(
          xg, agg, degp, w, b, skip, w_nz)


def _l4_call(y4, agg4p, degp, b4, z, w_cl):
  def body(y4_r, agg_r, degp_r, bt, z_r, wcl, y5_o, hz_o):
    i = pl.program_id(0)
    rdeg = _rdeg_of(degp_r[...])
    ag = agg_r[0, :, :D_Z] + agg_r[1, :, :D_Z]
    o = jnp.maximum(y4_r[..., :D_Z] + ag * rdeg[:, None] + bt[...], 0.0)
    o = o + jnp.where(i < U // R, 1.0, 0.0) * z_r[...]
    hz_o[...] = o
    y5 = _dot(o, wcl[...])
    y5_o[...] = jnp.concatenate([y5, jnp.zeros((R, 128 - D_CL), F32)], axis=1)

  return pl.pallas_call(
      body, grid=(N // R,),
      in_specs=[
          pl.BlockSpec((R, 128), lambda i: (i, 0)),
          pl.BlockSpec((2, R, 128), lambda i: (0, i, 0)),
          pl.BlockSpec((2, R, 128), lambda i: (0, i, 0)),
          pl.BlockSpec((D_Z,), lambda i: (0,)),
          pl.BlockSpec((R, D_Z), lambda i: (jnp.minimum(i, U // R - 1), 0)),
          pl.BlockSpec((D_Z, D_CL), lambda i: (0, 0)),
      ],
      out_specs=[
          pl.BlockSpec((R, 128), lambda i: (i, 0)),
          pl.BlockSpec((R, D_Z), lambda i: (i, 0)),
      ],
      out_shape=[
          jax.ShapeDtypeStruct((N, 128), F32),
          jax.ShapeDtypeStruct((N, D_Z), F32),
      ])(y4, agg4p, degp, b4, z, w_cl)


def _fin_call(y5, agg5p, degp, b5, hz, cl):
  def body(y5_r, agg_r, degp_r, bt, hz_r, cl_r, xo_o, q_o):
    rdeg = _rdeg_of(degp_r[...])
    ag = agg_r[0, :, :D_CL] + agg_r[1, :, :D_CL]
    o = y5_r[..., :D_CL] + ag * rdeg[:, None] + bt[...]
    xo_o[...] = jax.nn.sigmoid(o)
    hzb = hz_r[...]
    clm = cl_r[...]
    cols = []
    for kk in range(D_CL):
      diff = hzb - clm[kk][None, :]
      cols.append(jnp.sum(diff * diff, axis=1))
    d = jnp.stack(cols, axis=1)
    qv = 1.0 / (1.0 + d)
    q_o[...] = qv / jnp.sum(qv, axis=1, keepdims=True)

  return pl.pallas_call(
      body, grid=(U // R,),
      in_specs=[
          pl.BlockSpec((R, 128), lambda i: (i, 0)),
          pl.BlockSpec((2, R, 128), lambda i: (0, i, 0)),
          pl.BlockSpec((2, R, 128), lambda i: (0, i, 0)),
          pl.BlockSpec((D_CL,), lambda i: (0,)),
          pl.BlockSpec((R, D_Z), lambda i: (i, 0)),
          pl.BlockSpec((D_CL, D_Z), lambda i: (0, 0)),
      ],
      out_specs=[
          pl.BlockSpec((R, D_CL), lambda i: (i, 0)),
          pl.BlockSpec((R, D_CL), lambda i: (i, 0)),
      ],
      out_shape=[
          jax.ShapeDtypeStruct((U, D_CL), F32),
          jax.ShapeDtypeStruct((U, D_CL), F32),
      ])(y5, agg5p, degp, b5, hz, cl)


def kernel(x, edge_u_x, edge_u_id, edge_index,
           ae_enc_in_W, ae_enc_in_b, ae_henc0_W, ae_henc0_b,
           ae_henc1_W, ae_henc1_b, ae_z_W, ae_z_b,
           ae_dec_in_W, ae_dec_in_b, ae_hdec0_W, ae_hdec0_b,
           ae_hdec1_W, ae_hdec1_b, ae_xbar_W, ae_xbar_b,
           gnn_in_W, gnn_in_b, gnn_h0_W, gnn_h0_b, gnn_h1_W, gnn_h1_b,
           gnn_nz_W, gnn_nz_b, gnn_cl_W, gnn_cl_b, cluster_layer):
  src = edge_index[0]
  dst = edge_index[1]
  pad = EPAD - E
  src_p = jnp.concatenate([src, jnp.zeros((pad,), jnp.int32)])
  dst_p = jnp.concatenate([dst, jnp.full((pad,), N, jnp.int32)])
  src_off = src_p[None, :] + (jnp.arange(4, dtype=jnp.int32) * N)[:, None]
  src4r = src_off.reshape(4, 16, 80, 128)
  src32 = src_p.reshape(32, 40, 128)
  dst16 = dst_p.reshape(16, 80, 128)
  dst32 = dst_p.reshape(32, 40, 128)
  e0 = jnp.zeros((128, 128), F32).at[:, 0].set(1.0)
  z128 = jnp.zeros((640, 128), F32)

  degp = _DEG(dst32, e0, z128).reshape(2, NPAD, 128)

  ae_ws = (ae_enc_in_W, ae_enc_in_b, ae_henc0_W, ae_henc0_b,
           ae_henc1_W, ae_henc1_b, ae_z_W, ae_z_b,
           ae_dec_in_W, ae_dec_in_b, ae_hdec0_W, ae_hdec0_b,
           ae_hdec1_W, ae_hdec1_b, ae_xbar_W, ae_xbar_b)
  enc0, enc1, enc2, z_ae, x_bar = _ae_call(edge_u_x, ae_ws)

  xs1 = x.reshape(N, 2, 128).transpose(1, 0, 2).reshape(2 * N, 128)
  agg1 = _AGG2(xs1, src4r[:2], dst16, z128).reshape(2, NPAD, 128)
  xg1 = _l1_call(x, agg1, degp, gnn_in_W, gnn_in_b, enc0)

  agg2 = _AGG4(xg1.reshape(4 * N, 128), src4r, dst16, z128).reshape(
      4, NPAD, 128)
  xg2 = _lmid_call(xg1, agg2, degp, gnn_h0_W, gnn_h0_b, enc1)
  agg3 = _AGG4(xg2.reshape(4 * N, 128), src4r, dst16, z128).reshape(
      4, NPAD, 128)
  y4 = _l3_call(xg2, agg3, degp, gnn_h1_W, gnn_h1_b, enc2, gnn_nz_W)

  agg4 = _AGG1(y4, src32, dst32, z128).reshape(2, NPAD, 128)
  y5, hz_full = _l4_call(y4, agg4, degp, gnn_nz_b, z_ae, gnn_cl_W)
  hz = hz_full[:U]

  agg5 = _AGG1(y5, src32, dst32, z128).reshape(2, NPAD, 128)
  x_out, q = _fin_call(y5, agg5, degp, gnn_cl_b, hz, cluster_layer)

  return (x_out, x_bar, q)


# speedup vs baseline: 2.9425x; 1.0479x over previous
import functools

import jax
import jax.numpy as jnp
from jax import lax
from jax.experimental import pallas as pl
from jax.experimental.pallas import tpu as pltpu
from jax.experimental.pallas import tpu_sc as plsc

N = 10000
NPAD = 10240
E = 160000
EPAD = 163840
U = 4000
D_IN = 256
D_ENC = 512
D_Z = 64
D_CL = 32
R = 400
F32 = jnp.float32


_MESH = plsc.VectorSubcoreMesh(
    core_axis_name="c", subcore_axis_name="s", num_cores=2, num_subcores=16)


def _deg_kernel():
  @functools.partial(
      pl.kernel,
      out_type=jax.ShapeDtypeStruct((2 * NPAD, 128), F32),
      mesh=_MESH,
      scratch_types=[
          pltpu.VMEM_SHARED((NPAD, 128), F32),
          pltpu.VMEM((40, 128), jnp.int32),
          pltpu.VMEM((128, 128), F32),
          pltpu.SemaphoreType.DMA,
      ],
  )
  def k(dst32, e0, zeros128, out, acc, didx, gbuf, sem):
    cid = lax.axis_index("c")
    sid = lax.axis_index("s")
    w = cid * 16 + sid
    pltpu.sync_copy(e0, gbuf)
    pltpu.sync_copy(dst32.at[w], didx)
    pltpu.sync_copy(zeros128, acc.at[pl.ds(sid * 640, 640)])
    plsc.subcore_barrier()

    def body(j, carry):
      pltpu.sync_copy(gbuf, acc.at[didx.at[j]], add=True)
      return carry

    lax.fori_loop(0, 40, body, 0)
    plsc.subcore_barrier()
    pltpu.sync_copy(acc.at[pl.ds(sid * 640, 640)],
                    out.at[pl.ds(cid * NPAD + sid * 640, 640)])

  return k


def _agg_slice_kernel(C):
  halfc = C // 2

  @functools.partial(
      pl.kernel,
      out_type=jax.ShapeDtypeStruct((C * NPAD, 128), F32),
      mesh=_MESH,
      scratch_types=[
          pltpu.VMEM_SHARED((NPAD, 128), F32),
          pltpu.VMEM((40, 128), jnp.int32),
          pltpu.VMEM((40, 128), jnp.int32),
          pltpu.VMEM((2 * 128, 128), F32),
          [pltpu.SemaphoreType.DMA] * 2,
      ],
  )
  def k(table, srcr, dst16, zeros128, out, acc, sidx, didx, gbuf, gsem):
    cid = lax.axis_index("c")
    sid = lax.axis_index("s")
    for kk in range(halfc):
      sl = cid * halfc + kk
      pltpu.sync_copy(zeros128, acc.at[pl.ds(sid * 640, 640)])
      plsc.subcore_barrier()
      for half in range(2):
        pltpu.sync_copy(srcr.at[sl, sid, pl.ds(half * 40, 40)], sidx)
        pltpu.sync_copy(dst16.at[sid, pl.ds(half * 40, 40)], didx)

        def body(gi, carry):
          gds = [
              pltpu.async_copy(table.at[sidx.at[gi * 2 + b]],
                               gbuf.at[pl.ds(b * 128, 128)], gsem[b])
              for b in range(2)
          ]
          for b in range(2):
            gds[b].wait()
            pltpu.sync_copy(gbuf.at[pl.ds(b * 128, 128)],
                            acc.at[didx.at[gi * 2 + b]], add=True)
          return carry

        lax.fori_loop(0, 20, body, 0)
      plsc.subcore_barrier()
      pltpu.sync_copy(acc.at[pl.ds(sid * 640, 640)],
                      out.at[pl.ds(sl * NPAD + sid * 640, 640)])
      if kk + 1 < halfc:
        plsc.subcore_barrier()

  return k


def _agg_narrow_kernel():
  W = 128

  @functools.partial(
      pl.kernel,
      out_type=jax.ShapeDtypeStruct((2 * NPAD, W), F32),
      mesh=_MESH,
      scratch_types=[
          pltpu.VMEM_SHARED((NPAD, W), F32),
          pltpu.VMEM((40, 128), jnp.int32),
          pltpu.VMEM((40, 128), jnp.int32),
          pltpu.VMEM((2 * 128, W), F32),
          [pltpu.SemaphoreType.DMA] * 2,
      ],
  )
  def k(table, src32, dst32, zerosw, out, acc, sidx, didx, gbuf, gsem):
    cid = lax.axis_index("c")
    sid = lax.axis_index("s")
    w = cid * 16 + sid
    pltpu.sync_copy(src32.at[w], sidx)
    pltpu.sync_copy(dst32.at[w], didx)
    pltpu.sync_copy(zerosw, acc.at[pl.ds(sid * 640, 640)])
    plsc.subcore_barrier()

    def body(gi, carry):
      gds = [
          pltpu.async_copy(table.at[sidx.at[gi * 2 + b]],
                           gbuf.at[pl.ds(b * 128, 128)], gsem[b])
          for b in range(2)
      ]
      for b in range(2):
        gds[b].wait()
        pltpu.sync_copy(gbuf.at[pl.ds(b * 128, 128)],
                        acc.at[didx.at[gi * 2 + b]], add=True)
      return carry

    lax.fori_loop(0, 20, body, 0)
    plsc.subcore_barrier()
    pltpu.sync_copy(acc.at[pl.ds(sid * 640, 640)],
                    out.at[pl.ds(cid * NPAD + sid * 640, 640)])

  return k


_DEG = _deg_kernel()
_AGG2 = _agg_slice_kernel(2)
_AGG4 = _agg_slice_kernel(4)
_AGG1 = _agg_narrow_kernel()


def _dot(a, b):
  return jnp.dot(a, b, preferred_element_type=F32)


def _ae_body(exu, w0, b0, w1, b1, w2, b2, wz, bz, wd, bd, wh0, bh0, wh1, bh1,
             wx, bx, enc0_o, enc1_o, enc2_o, z_o, xbar_o):
  e0 = jnp.maximum(_dot(exu[...], w0[...]) + b0[...], 0.0)
  e1 = jnp.maximum(_dot(e0, w1[...]) + b1[...], 0.0)
  e2 = jnp.maximum(_dot(e1, w2[...]) + b2[...], 0.0)
  z = _dot(e2, wz[...]) + bz[...]
  d0 = jnp.maximum(_dot(z, wd[...]) + bd[...], 0.0)
  d1 = jnp.maximum(_dot(d0, wh0[...]) + bh0[...], 0.0)
  d2 = jnp.maximum(_dot(d1, wh1[...]) + bh1[...], 0.0)
  xb = _dot(d2, wx[...]) + bx[...]
  enc0_o[...] = e0
  enc1_o[...] = e1
  enc2_o[...] = e2
  z_o[...] = z
  xbar_o[...] = xb


def _ae_call(exu, ws):
  full = lambda shp: pl.BlockSpec(shp, lambda i: (0,) * len(shp))
  specs = [pl.BlockSpec((R, D_IN), lambda i: (i, 0))]
  for wname, din, dout in [
      ("ae_enc_in", D_IN, D_ENC), ("ae_henc0", D_ENC, D_ENC),
      ("ae_henc1", D_ENC, D_ENC), ("ae_z", D_ENC, D_Z),
      ("ae_dec_in", D_Z, D_ENC), ("ae_hdec0", D_ENC, D_ENC),
      ("ae_hdec1", D_ENC, D_ENC), ("ae_xbar", D_ENC, D_IN)]:
    specs.append(full((din, dout)))
    specs.append(full((dout,)))
  outs = [
      jax.ShapeDtypeStruct((U, D_ENC), F32),
      jax.ShapeDtypeStruct((U, D_ENC), F32),
      jax.ShapeDtypeStruct((U, D_ENC), F32),
      jax.ShapeDtypeStruct((U, D_Z), F32),
      jax.ShapeDtypeStruct((U, D_IN), F32),
  ]
  out_specs = [
      pl.BlockSpec((R, D_ENC), lambda i: (i, 0)),
      pl.BlockSpec((R, D_ENC), lambda i: (i, 0)),
      pl.BlockSpec((R, D_ENC), lambda i: (i, 0)),
      pl.BlockSpec((R, D_Z), lambda i: (i, 0)),
      pl.BlockSpec((R, D_IN), lambda i: (i, 0)),
  ]
  return pl.pallas_call(
      _ae_body, grid=(U // R,), in_specs=specs, out_specs=out_specs,
      out_shape=outs)(exu, *ws)


def _rdeg_of(degp):
  deg = degp[0, :, 0] + degp[1, :, 0]
  return 1.0 / jnp.maximum(deg, 1.0)


def _l1_call(x, agg1, degp, w, b, enc0):
  def body(x_ref, agg, degp_r, wt, bt, skip, out):
    i = pl.program_id(0)
    rdeg = _rdeg_of(degp_r[...])
    a = jnp.concatenate([agg[0], agg[1]], axis=1)
    o = jnp.maximum(
        _dot(x_ref[...] + a * rdeg[:, None], wt[...]) + bt[...], 0.0)
    o = o + jnp.where(i < U // R, 1.0, 0.0) * skip[...]
    for c in range(4):
      out[c] = o[:, c * 128:(c + 1) * 128]

  return pl.pallas_call(
      body, grid=(N // R,),
      in_specs=[
          pl.BlockSpec((R, D_IN), lambda i: (i, 0)),
          pl.BlockSpec((2, R, 128), lambda i: (0, i, 0)),
          pl.BlockSpec((2, R, 128), lambda i: (0, i, 0)),
          pl.BlockSpec((D_IN, D_ENC), lambda i: (0, 0)),
          pl.BlockSpec((D_ENC,), lambda i: (0,)),
          pl.BlockSpec((R, D_ENC), lambda i: (jnp.minimum(i, U // R - 1), 0)),
      ],
      out_specs=pl.BlockSpec((4, R, 128), lambda i: (0, i, 0)),
      out_shape=jax.ShapeDtypeStruct((4, N, 128), F32))(
          x, agg1, degp, w, b, enc0)


def _lmid_call(xg, agg, degp, w, b, skip):
  def body(x_refs, agg_r, degp_r, wt, bt, skip_r, out):
    i = pl.program_id(0)
    rdeg = _rdeg_of(degp_r[...])
    h = jnp.concatenate([x_refs[c] for c in range(4)], axis=1)
    a = jnp.concatenate([agg_r[c] for c in range(4)], axis=1)
    o = jnp.maximum(_dot(h + a * rdeg[:, None], wt[...]) + bt[...], 0.0)
    o = o + jnp.where(i < U // R, 1.0, 0.0) * skip_r[...]
    for c in range(4):
      out[c] = o[:, c * 128:(c + 1) * 128]

  return pl.pallas_call(
      body, grid=(N // R,),
      in_specs=[
          pl.BlockSpec((4, R, 128), lambda i: (0, i, 0)),
          pl.BlockSpec((4, R, 128), lambda i: (0, i, 0)),
          pl.BlockSpec((2, R, 128), lambda i: (0, i, 0)),
          pl.BlockSpec((D_ENC, D_ENC), lambda i: (0, 0)),
          pl.BlockSpec((D_ENC,), lambda i: (0,)),
          pl.BlockSpec((R, D_ENC), lambda i: (jnp.minimum(i, U // R - 1), 0)),
      ],
      out_specs=pl.BlockSpec((4, R, 128), lambda i: (0, i, 0)),
      out_shape=jax.ShapeDtypeStruct((4, N, 128), F32))(
          xg, agg, degp, w, b, skip)


def _l3_call(xg, agg, degp, w, b, skip, w_nz):
  def body(x_refs, agg_r, degp_r, wt, bt, skip_r, wnz, out):
    i = pl.program_id(0)
    rdeg = _rdeg_of(degp_r[...])
    h = jnp.concatenate([x_refs[c] for c in range(4)], axis=1)
    a = jnp.concatenate([agg_r[c] for c in range(4)], axis=1)
    o = jnp.maximum(_dot(h + a * rdeg[:, None], wt[...]) + bt[...], 0.0)
    o = o + jnp.where(i < U // R, 1.0, 0.0) * skip_r[...]
    y4 = _dot(o, wnz[...])
    out[...] = jnp.concatenate([y4, jnp.zeros((R, 128 - D_Z), F32)], axis=1)

  return pl.pallas_call(
      body, grid=(N // R,),
      in_specs=[
          pl.BlockSpec((4, R, 128), lambda i: (0, i, 0)),
          pl.BlockSpec((4, R, 128), lambda i: (0, i, 0)),
          pl.BlockSpec((2, R, 128), lambda i: (0, i, 0)),
          pl.BlockSpec((D_ENC, D_ENC), lambda i: (0, 0)),
          pl.BlockSpec((D_ENC,), lambda i: (0,)),
          pl.BlockSpec((R, D_ENC), lambda i: (jnp.minimum(i, U // R - 1), 0)),
          pl.BlockSpec((D_ENC, D_Z), lambda i: (0, 0)),
      ],
      out_specs=pl.BlockSpec((R, 128), lambda i: (i, 0)),
      out_shape=jax.ShapeDtypeStruct((N, 128), F32))(
          xg, agg, degp, w, b, skip, w_nz)


def _l4_call(y4, agg4p, degp, b4, z, w_cl):
  def body(y4_r, agg_r, degp_r, bt, z_r, wcl, y5_o, hz_o):
    i = pl.program_id(0)
    rdeg = _rdeg_of(degp_r[...])
    ag = agg_r[0, :, :D_Z] + agg_r[1, :, :D_Z]
    o = jnp.maximum(y4_r[..., :D_Z] + ag * rdeg[:, None] + bt[...], 0.0)
    o = o + jnp.where(i < U // R, 1.0, 0.0) * z_r[...]
    hz_o[...] = o
    y5 = _dot(o, wcl[...])
    y5_o[...] = jnp.concatenate([y5, jnp.zeros((R, 128 - D_CL), F32)], axis=1)

  return pl.pallas_call(
      body, grid=(N // R,),
      in_specs=[
          pl.BlockSpec((R, 128), lambda i: (i, 0)),
          pl.BlockSpec((2, R, 128), lambda i: (0, i, 0)),
          pl.BlockSpec((2, R, 128), lambda i: (0, i, 0)),
          pl.BlockSpec((D_Z,), lambda i: (0,)),
          pl.BlockSpec((R, D_Z), lambda i: (jnp.minimum(i, U // R - 1), 0)),
          pl.BlockSpec((D_Z, D_CL), lambda i: (0, 0)),
      ],
      out_specs=[
          pl.BlockSpec((R, 128), lambda i: (i, 0)),
          pl.BlockSpec((R, D_Z), lambda i: (i, 0)),
      ],
      out_shape=[
          jax.ShapeDtypeStruct((N, 128), F32),
          jax.ShapeDtypeStruct((N, D_Z), F32),
      ])(y4, agg4p, degp, b4, z, w_cl)


def _fin_call(y5, agg5p, degp, b5, hz, cl):
  def body(y5_r, agg_r, degp_r, bt, hz_r, cl_r, xo_o, q_o):
    rdeg = _rdeg_of(degp_r[...])
    ag = agg_r[0, :, :D_CL] + agg_r[1, :, :D_CL]
    o = y5_r[..., :D_CL] + ag * rdeg[:, None] + bt[...]
    xo_o[...] = jax.nn.sigmoid(o)
    hzb = hz_r[...]
    clm = cl_r[...]
    cols = []
    for kk in range(D_CL):
      diff = hzb - clm[kk][None, :]
      cols.append(jnp.sum(diff * diff, axis=1))
    d = jnp.stack(cols, axis=1)
    qv = 1.0 / (1.0 + d)
    q_o[...] = qv / jnp.sum(qv, axis=1, keepdims=True)

  return pl.pallas_call(
      body, grid=(U // R,),
      in_specs=[
          pl.BlockSpec((R, 128), lambda i: (i, 0)),
          pl.BlockSpec((2, R, 128), lambda i: (0, i, 0)),
          pl.BlockSpec((2, R, 128), lambda i: (0, i, 0)),
          pl.BlockSpec((D_CL,), lambda i: (0,)),
          pl.BlockSpec((R, D_Z), lambda i: (i, 0)),
          pl.BlockSpec((D_CL, D_Z), lambda i: (0, 0)),
      ],
      out_specs=[
          pl.BlockSpec((R, D_CL), lambda i: (i, 0)),
          pl.BlockSpec((R, D_CL), lambda i: (i, 0)),
      ],
      out_shape=[
          jax.ShapeDtypeStruct((U, D_CL), F32),
          jax.ShapeDtypeStruct((U, D_CL), F32),
      ])(y5, agg5p, degp, b5, hz, cl)


def kernel(x, edge_u_x, edge_u_id, edge_index,
           ae_enc_in_W, ae_enc_in_b, ae_henc0_W, ae_henc0_b,
           ae_henc1_W, ae_henc1_b, ae_z_W, ae_z_b,
           ae_dec_in_W, ae_dec_in_b, ae_hdec0_W, ae_hdec0_b,
           ae_hdec1_W, ae_hdec1_b, ae_xbar_W, ae_xbar_b,
           gnn_in_W, gnn_in_b, gnn_h0_W, gnn_h0_b, gnn_h1_W, gnn_h1_b,
           gnn_nz_W, gnn_nz_b, gnn_cl_W, gnn_cl_b, cluster_layer):
  src = edge_index[0]
  dst = edge_index[1]
  pad = EPAD - E
  src_p = jnp.concatenate([src, jnp.zeros((pad,), jnp.int32)])
  dst_p = jnp.concatenate([dst, jnp.full((pad,), N, jnp.int32)])
  src_off = src_p[None, :] + (jnp.arange(4, dtype=jnp.int32) * N)[:, None]
  src4r = src_off.reshape(4, 16, 80, 128)
  src32 = src_p.reshape(32, 40, 128)
  dst16 = dst_p.reshape(16, 80, 128)
  dst32 = dst_p.reshape(32, 40, 128)
  e0 = jnp.zeros((128, 128), F32).at[:, 0].set(1.0)
  z128 = jnp.zeros((640, 128), F32)

  degp = _DEG(dst32, e0, z128).reshape(2, NPAD, 128)

  ae_ws = (ae_enc_in_W, ae_enc_in_b, ae_henc0_W, ae_henc0_b,
           ae_henc1_W, ae_henc1_b, ae_z_W, ae_z_b,
           ae_dec_in_W, ae_dec_in_b, ae_hdec0_W, ae_hdec0_b,
           ae_hdec1_W, ae_hdec1_b, ae_xbar_W, ae_xbar_b)
  enc0, enc1, enc2, z_ae, x_bar = _ae_call(edge_u_x, ae_ws)

  xs1 = x.reshape(N, 2, 128).transpose(1, 0, 2).reshape(2 * N, 128)
  agg1 = _AGG2(xs1, src4r[:2], dst16, z128).reshape(2, NPAD, 128)
  xg1 = _l1_call(x, agg1, degp, gnn_in_W, gnn_in_b, enc0)

  agg2 = _AGG4(xg1.reshape(4 * N, 128), src4r, dst16, z128).reshape(
      4, NPAD, 128)
  xg2 = _lmid_call(xg1, agg2, degp, gnn_h0_W, gnn_h0_b, enc1)
  agg3 = _AGG4(xg2.reshape(4 * N, 128), src4r, dst16, z128).reshape(
      4, NPAD, 128)
  y4 = _l3_call(xg2, agg3, degp, gnn_h1_W, gnn_h1_b, enc2, gnn_nz_W)

  agg4 = _AGG1(y4, src32, dst32, z128).reshape(2, NPAD, 128)
  y5, hz_full = _l4_call(y4, agg4, degp, gnn_nz_b, z_ae, gnn_cl_W)
  hz = hz_full[:U]

  agg5 = _AGG1(y5, src32, dst32, z128).reshape(2, NPAD, 128)
  x_out, q = _fin_call(y5, agg5, degp, gnn_cl_b, hz, cluster_layer)

  return (x_out, x_bar, q)
